# baseline (speedup 1.0000x reference)
import jax
import jax.numpy as jnp
from jax.experimental import pallas as pl

_NP = 4096
_P = 64
_BM = 64


def _chamfer_body(px_ref, py_ref, pz_ref, tx_ref, ty_ref, tz_ref, out_ref):
    @pl.when(pl.program_id(0) == 0)
    def _init():
        out_ref[...] = jnp.zeros_like(out_ref)

    px = px_ref[...]
    py = py_ref[...]
    pz = pz_ref[...]
    tx = tx_ref[...]
    ty = ty_ref[...]
    tz = tz_ref[...]

    dx = px[:, :, None] - tx[:, None, :]
    d2 = dx * dx
    dy = py[:, :, None] - ty[:, None, :]
    d2 = d2 + dy * dy
    dz = pz[:, :, None] - tz[:, None, :]
    d2 = d2 + dz * dz

    fwd = jnp.min(d2, axis=2)
    bwd = jnp.min(d2, axis=1)
    out_ref[...] += (jnp.sum(fwd) + jnp.sum(bwd)).reshape(1, 1)


def kernel(pred_patches, target_patches):
    pred = pred_patches.reshape(_NP, _P, 3).transpose(2, 0, 1)
    tgt = target_patches.reshape(_NP, _P, 3).transpose(2, 0, 1)

    plane = pl.BlockSpec((_BM, _P), lambda i: (i, 0))
    total = pl.pallas_call(
        _chamfer_body,
        grid=(_NP // _BM,),
        in_specs=[plane] * 6,
        out_specs=pl.BlockSpec((1, 1), lambda i: (0, 0)),
        out_shape=jax.ShapeDtypeStruct((1, 1), jnp.float32),
    )(pred[0], pred[1], pred[2], tgt[0], tgt[1], tgt[2])

    return total[0, 0] * (1.0 / (_NP * _P))
